# baseline (speedup 1.0000x reference)
import jax
import jax.numpy as jnp
from jax import lax
from jax.experimental import pallas as pl
from jax.experimental.pallas import tpu as pltpu
from jax.experimental.pallas import tpu_sc as plsc

_NC = 2
_NS = 16
_NW = _NC * _NS
_L = 16

_E = 320000
_D = 128
_W = _D // 2
_Q = _W // _L
_EW = _E // _NW
_C = 80
_G = _EW // _C
_NB = 4


def _sc_body(z_hbm, src_hbm, dst_hbm, out_hbm,
             idx_s, idx_d, table, rows_s, rows_d, out_all, t_ref,
             sem_s0, sem_s1, sem_s2, sem_s3,
             sem_d0, sem_d1, sem_d2, sem_d3):
    sid = lax.axis_index("s")
    wid = sid * _NC + lax.axis_index("c")
    base = wid * _EW

    row_iota = lax.iota(jnp.int32, _L)
    flat_iota = row_iota * _L

    _R = 10000 // _NS
    pltpu.sync_copy(z_hbm.at[pl.ds(sid * _R, _R)],
                    table.at[pl.ds(sid * _R, _R)])
    plsc.subcore_barrier()

    pltpu.sync_copy(src_hbm.at[pl.ds(base, _EW)], idx_s)
    pltpu.sync_copy(dst_hbm.at[pl.ds(base, _EW)], idx_d)

    sems_s = (sem_s0, sem_s1, sem_s2, sem_s3)
    sems_d = (sem_d0, sem_d1, sem_d2, sem_d3)

    def src_for(b):
        return z_hbm if b % 2 == 0 else table

    def start(g, b):
        pltpu.async_copy(src_for(b).at[idx_s.at[pl.ds(g * _C, _C)]],
                         rows_s.at[b], sems_s[b])
        pltpu.async_copy(src_for(b).at[idx_d.at[pl.ds(g * _C, _C)]],
                         rows_d.at[b], sems_d[b])

    def wait(b):
        pltpu.make_async_copy(src_for(b).at[idx_s.at[pl.ds(0, _C)]],
                              rows_s.at[b], sems_s[b]).wait()
        pltpu.make_async_copy(src_for(b).at[idx_d.at[pl.ds(0, _C)]],
                              rows_d.at[b], sems_d[b]).wait()

    def compute(g, b):
        rs = rows_s.at[b]
        rd = rows_d.at[b]

        @plsc.parallel_loop(0, _C // _L, step=1, unroll=1)
        def blk(k):
            e0 = k * _L
            tk = t_ref.at[k]
            accs = [None] * _L
            for q in range(_Q):
                for j in range(_L):
                    ws = rs[e0 + j, pl.ds(q * _L, _L)]
                    wd = rd[e0 + j, pl.ds(q * _L, _L)]
                    ps = (plsc.bitcast(ws, jnp.bfloat16)
                          * plsc.bitcast(wd, jnp.bfloat16))
                    pa, pb = plsc.unpack(ps, format=plsc.PackFormat.INTERLEAVED)
                    term = pa + pb
                    accs[j] = term if accs[j] is None else accs[j] + term
            for j in range(_L):
                tk[pl.ds(j * _L, _L)] = accs[j]
            res = plsc.load_gather(tk, [flat_iota])
            for d in range(1, _L):
                res = res + plsc.load_gather(tk, [flat_iota + d])
            e = jnp.exp(-jnp.abs(res))
            a = 1.0 / (1.0 + e)
            out_all[pl.ds(g * _C + k * _L, _L)] = jnp.where(res >= 0, a, 1.0 - a)

    for b in range(_NB - 1):
        start(b, b)

    def step4(gg, carry):
        for b in range(_NB):
            g = gg + b
            wait(b)

            @pl.when(g + (_NB - 1) < _G)
            def _():
                start(g + (_NB - 1), (b + _NB - 1) % _NB)

            compute(g, b)
        return carry

    lax.fori_loop(0, (_G - 1) // _NB, lambda i, c: step4(i * _NB, c), 0)
    wait(0)
    compute(_G - 1, 0)

    pltpu.sync_copy(out_all, out_hbm.at[pl.ds(base, _EW)])


@jax.jit
def _run(z32, src, dst):
    mesh = plsc.VectorSubcoreMesh(core_axis_name="c", subcore_axis_name="s")
    f = pl.kernel(
        _sc_body,
        out_type=jax.ShapeDtypeStruct((_E,), jnp.float32),
        mesh=mesh,
        compiler_params=pltpu.CompilerParams(needs_layout_passes=False,
                                             use_tc_tiling_on_sc=False),
        scratch_types=[
            pltpu.VMEM((_EW,), jnp.int32),
            pltpu.VMEM((_EW,), jnp.int32),
            pltpu.VMEM_SHARED((10000, _W), jnp.int32),
            pltpu.VMEM((_NB, _C, _W), jnp.int32),
            pltpu.VMEM((_NB, _C, _W), jnp.int32),
            pltpu.VMEM((_EW,), jnp.float32),
            pltpu.VMEM((_C // _L, _L * _L), jnp.float32),
            pltpu.SemaphoreType.DMA,
            pltpu.SemaphoreType.DMA,
            pltpu.SemaphoreType.DMA,
            pltpu.SemaphoreType.DMA,
            pltpu.SemaphoreType.DMA,
            pltpu.SemaphoreType.DMA,
            pltpu.SemaphoreType.DMA,
            pltpu.SemaphoreType.DMA,
        ],
    )
    return f(z32, src, dst)


def kernel(z, edge_index):
    zb = z.astype(jnp.bfloat16)
    z32 = lax.bitcast_convert_type(zb.reshape(-1, _W, 2), jnp.int32)
    src = edge_index[0].astype(jnp.int32)
    dst = edge_index[1].astype(jnp.int32)
    return _run(z32, src, dst)


# speedup vs baseline: 1.0236x; 1.0236x over previous
import jax
import jax.numpy as jnp
from jax import lax
from jax.experimental import pallas as pl
from jax.experimental.pallas import tpu as pltpu
from jax.experimental.pallas import tpu_sc as plsc

_NC = 2
_NS = 16
_NW = _NC * _NS
_L = 16

_E = 320000
_D = 128
_W = _D // 2
_Q = _W // _L
_EW = _E // _NW
_C = 80
_G = _EW // _C
_NB = 4


def _sc_body(z_hbm, src_hbm, dst_hbm, out_hbm,
             idx_s, idx_d, rows_s, rows_d, out_all, t_ref,
             sem_s0, sem_s1, sem_s2, sem_s3,
             sem_d0, sem_d1, sem_d2, sem_d3):
    wid = lax.axis_index("s") * _NC + lax.axis_index("c")
    base = wid * _EW

    row_iota = lax.iota(jnp.int32, _L)
    flat_iota = row_iota * _L

    pltpu.sync_copy(src_hbm.at[pl.ds(base, _EW)], idx_s)
    pltpu.sync_copy(dst_hbm.at[pl.ds(base, _EW)], idx_d)

    sems_s = (sem_s0, sem_s1, sem_s2, sem_s3)
    sems_d = (sem_d0, sem_d1, sem_d2, sem_d3)

    def start(g, b):
        pltpu.async_copy(z_hbm.at[idx_s.at[pl.ds(g * _C, _C)]],
                         rows_s.at[b], sems_s[b])
        pltpu.async_copy(z_hbm.at[idx_d.at[pl.ds(g * _C, _C)]],
                         rows_d.at[b], sems_d[b])

    def wait(b):
        pltpu.make_async_copy(z_hbm.at[idx_s.at[pl.ds(0, _C)]],
                              rows_s.at[b], sems_s[b]).wait()
        pltpu.make_async_copy(z_hbm.at[idx_d.at[pl.ds(0, _C)]],
                              rows_d.at[b], sems_d[b]).wait()

    def compute(g, b):
        rs = rows_s.at[b]
        rd = rows_d.at[b]

        @plsc.parallel_loop(0, _C // _L, step=1, unroll=1)
        def blk(k):
            e0 = k * _L
            tk = t_ref.at[k]
            accs = [None] * _L
            for q in range(_Q):
                for j in range(_L):
                    ws = rs[e0 + j, pl.ds(q * _L, _L)]
                    wd = rd[e0 + j, pl.ds(q * _L, _L)]
                    ps = (plsc.bitcast(ws, jnp.bfloat16)
                          * plsc.bitcast(wd, jnp.bfloat16))
                    pa, pb = plsc.unpack(ps, format=plsc.PackFormat.INTERLEAVED)
                    term = pa + pb
                    accs[j] = term if accs[j] is None else accs[j] + term
            for j in range(_L):
                tk[pl.ds(j * _L, _L)] = accs[j]
            res = plsc.load_gather(tk, [flat_iota])
            for d in range(1, _L):
                res = res + plsc.load_gather(tk, [flat_iota + d])
            e = jnp.exp(-jnp.abs(res))
            a = 1.0 / (1.0 + e)
            out_all[pl.ds(g * _C + k * _L, _L)] = jnp.where(res >= 0, a, 1.0 - a)

    for b in range(_NB - 1):
        start(b, b)

    def step4(gg, carry):
        for b in range(_NB):
            g = gg + b
            wait(b)

            @pl.when(g + (_NB - 1) < _G)
            def _():
                start(g + (_NB - 1), (b + _NB - 1) % _NB)

            compute(g, b)
        return carry

    lax.fori_loop(0, (_G - 1) // _NB, lambda i, c: step4(i * _NB, c), 0)
    wait(0)
    compute(_G - 1, 0)

    pltpu.sync_copy(out_all, out_hbm.at[pl.ds(base, _EW)])


@jax.jit
def _run(z32, src, dst):
    mesh = plsc.VectorSubcoreMesh(core_axis_name="c", subcore_axis_name="s")
    f = pl.kernel(
        _sc_body,
        out_type=jax.ShapeDtypeStruct((_E,), jnp.float32),
        mesh=mesh,
        compiler_params=pltpu.CompilerParams(needs_layout_passes=False,
                                             use_tc_tiling_on_sc=False),
        scratch_types=[
            pltpu.VMEM((_EW,), jnp.int32),
            pltpu.VMEM((_EW,), jnp.int32),
            pltpu.VMEM((_NB, _C, _W), jnp.int32),
            pltpu.VMEM((_NB, _C, _W), jnp.int32),
            pltpu.VMEM((_EW,), jnp.float32),
            pltpu.VMEM((_C // _L, _L * _L), jnp.float32),
            pltpu.SemaphoreType.DMA,
            pltpu.SemaphoreType.DMA,
            pltpu.SemaphoreType.DMA,
            pltpu.SemaphoreType.DMA,
            pltpu.SemaphoreType.DMA,
            pltpu.SemaphoreType.DMA,
            pltpu.SemaphoreType.DMA,
            pltpu.SemaphoreType.DMA,
        ],
    )
    return f(z32, src, dst)


def kernel(z, edge_index):
    zb = z.astype(jnp.bfloat16)
    z32 = lax.bitcast_convert_type(zb.reshape(-1, _W, 2), jnp.int32)
    src = edge_index[0].astype(jnp.int32)
    dst = edge_index[1].astype(jnp.int32)
    return _run(z32, src, dst)
